# baseline (speedup 1.0000x reference)
import functools
import math

import jax
import jax.numpy as jnp
from jax import lax
from jax.experimental import pallas as pl
from jax.experimental.pallas import tpu as pltpu
from jax.experimental.pallas import tpu_sc as plsc

NC = 2
NS = 16
NW = NC * NS
L = 16
C = 64


NBUF = 4
DG = 2
DS = NBUF - DG
SLAB = 8
SUPER = 2


def _sc_aggregate(h_pad, sd_idx, n_pad, d, chunks_per_worker):
    rows_per_tile = n_pad // NS
    zchunks = rows_per_tile // C
    cpw = chunks_per_worker
    mesh = plsc.VectorSubcoreMesh(core_axis_name="c", subcore_axis_name="s")

    @functools.partial(
        pl.kernel,
        out_type=(
            jax.ShapeDtypeStruct((NC, n_pad, d), jnp.float32),
            jax.ShapeDtypeStruct((NW, n_pad), jnp.float32),
        ),
        mesh=mesh,
        compiler_params=pltpu.CompilerParams(needs_layout_passes=False),
        scratch_types=[
            [pltpu.VMEM((SLAB, 2, C), jnp.int32) for _ in range(2)],
            [pltpu.VMEM((C, d), jnp.float32) for _ in range(NBUF)],
            pltpu.VMEM((n_pad,), jnp.float32),
            pltpu.VMEM_SHARED((n_pad, d), jnp.float32),
            [pltpu.SemaphoreType.DMA for _ in range(NBUF)],
            [pltpu.SemaphoreType.DMA for _ in range(NBUF)],
            [pltpu.SemaphoreType.DMA for _ in range(2)],
        ],
    )
    def agg(h_ref, sd_ref, feat_ref, deg_ref, slab, rows, degs, acc,
            gsem, ssem, lsem):
        cid = lax.axis_index("c")
        sid = lax.axis_index("s")
        wid = cid * NS + sid

        zvec = jnp.zeros((L,), jnp.float32)

        def zrow(i, carry):
            for j in range(d // L):
                rows[0][i, pl.ds(j * L, L)] = zvec
            return carry

        lax.fori_loop(0, C, zrow, 0)

        def zdeg(i, carry):
            for j in range(8):
                degs[pl.ds(i * 8 * L + j * L, L)] = zvec
            return carry

        lax.fori_loop(0, n_pad // (8 * L), zdeg, 0)

        r0 = sid * rows_per_tile

        def zacc(k, carry):
            pltpu.sync_copy(rows[0], acc.at[pl.ds(r0 + k * C, C)])
            return carry

        lax.fori_loop(0, zchunks, zacc, 0)
        plsc.subcore_barrier()

        ones16 = jnp.full((L,), 1.0, jnp.float32)
        nslab = cpw // SLAB

        def gather(j_static_slab, k, b):
            pltpu.async_copy(h_ref.at[slab[j_static_slab].at[k, 0]],
                             rows[b], gsem[b])

        pltpu.async_copy(sd_ref.at[wid, 0], slab[0], lsem[0])
        pltpu.make_async_copy(sd_ref.at[wid, 0], slab[0], lsem[0]).wait()
        for jj in range(DG):
            gather(0, jj, jj % NBUF)

        def super_body(p, carry):
            for half in range(SUPER):
                s = p * SUPER + half
                for k in range(SLAB):
                    j = (p * SUPER + half) * SLAB + k
                    b = (SLAB * half + k) % NBUF
                    bg = (SLAB * half + k + DG) % NBUF

                    pltpu.make_async_copy(
                        h_ref.at[slab[half].at[k, 0]], rows[b], gsem[b]
                    ).wait()
                    pltpu.async_copy(rows[b], acc.at[slab[half].at[k, 1]],
                                     ssem[b], add=True)
                    for g in range(C // L):
                        dvec = slab[half][k, 1, pl.ds(g * L, L)]
                        plsc.addupdate_scatter(degs, [dvec], ones16)

                    if k == 2:
                        @pl.when(s + 1 < nslab)
                        def _():
                            pltpu.async_copy(sd_ref.at[wid, s + 1],
                                             slab[1 - half], lsem[1 - half])

                    @pl.when(j >= DS)
                    def _():
                        pltpu.make_async_copy(
                            rows[bg], acc.at[slab[half].at[k, 1]], ssem[bg]
                        ).wait()

                    if k == SLAB - DG:
                        @pl.when(s + 1 < nslab)
                        def _():
                            pltpu.make_async_copy(sd_ref.at[wid, s + 1],
                                                  slab[1 - half],
                                                  lsem[1 - half]).wait()
                    if k < SLAB - DG:
                        gather(half, k + DG, bg)
                    else:
                        @pl.when(s + 1 < nslab)
                        def _():
                            gather(1 - half, k + DG - SLAB, bg)
            return carry

        lax.fori_loop(0, nslab // SUPER, super_body, 0)

        for j in range(cpw - DS, cpw):
            b = j % NBUF
            pltpu.make_async_copy(
                rows[b], acc.at[slab[1].at[SLAB - 1, 1]], ssem[b]
            ).wait()
        plsc.subcore_barrier()

        pltpu.sync_copy(acc.at[pl.ds(r0, rows_per_tile)],
                        feat_ref.at[cid, pl.ds(r0, rows_per_tile)])
        pltpu.sync_copy(degs, deg_ref.at[wid])

    return agg(h_pad, sd_idx)


def _tc_finish(partials, deg_all, h_self, W_self, W_neigh, bm):
    n, d = h_self.shape
    out = W_self.shape[1]

    def body(p_ref, dg_ref, hs_ref, ws_ref, wn_ref, o_ref):
        deg = jnp.sum(dg_ref[...], axis=1, keepdims=True)
        neigh = (p_ref[0] + p_ref[1]) / jnp.maximum(deg, 1.0)
        z = (jnp.dot(hs_ref[...], ws_ref[...], preferred_element_type=jnp.float32)
             + jnp.dot(neigh, wn_ref[...], preferred_element_type=jnp.float32))
        z = jnp.maximum(z, 0.0)
        nrm = jnp.sqrt(jnp.sum(z * z, axis=1, keepdims=True))
        nrm = jnp.where(nrm == 0.0, 1.0, nrm)
        o_ref[...] = z / nrm

    return pl.pallas_call(
        body,
        grid=(n // bm,),
        in_specs=[
            pl.BlockSpec((NC, bm, d), lambda i: (0, i, 0)),
            pl.BlockSpec((bm, NW), lambda i: (i, 0)),
            pl.BlockSpec((bm, d), lambda i: (i, 0)),
            pl.BlockSpec((d, out), lambda i: (0, 0)),
            pl.BlockSpec((d, out), lambda i: (0, 0)),
        ],
        out_specs=pl.BlockSpec((bm, out), lambda i: (i, 0)),
        out_shape=jax.ShapeDtypeStruct((n, out), jnp.float32),
    )(partials, deg_all, h_self, W_self, W_neigh)


def kernel(h_neigh, h_self, edge_index, W_neigh, W_self):
    n, d = h_neigh.shape
    e = edge_index.shape[1]

    n_pad = ((n + 1 + NS * C - 1) // (NS * C)) * (NS * C)
    step = SLAB * SUPER
    cpw = ((math.ceil(e / (NW * C)) + step - 1) // step) * step
    e_pad = NW * C * cpw

    src = edge_index[0].astype(jnp.int32)
    dst = edge_index[1].astype(jnp.int32)
    pad = e_pad - e
    if pad:
        ar = jnp.arange(pad, dtype=jnp.int32)
        src = jnp.concatenate([src, ar % n])
        dst = jnp.concatenate([dst, n + ar % (n_pad - n)])
    sd = jnp.stack([src.reshape(NW, cpw, C), dst.reshape(NW, cpw, C)],
                   axis=2).reshape(NW, cpw // SLAB, SLAB, 2, C)

    partials, deg_all = _sc_aggregate(h_neigh, sd, n_pad, d, cpw)

    bm = next(b for b in (400, 200, 100, 50, 25, 10, 5, 1)
              if n % b == 0 and (b % 8 == 0 or b == n))
    return _tc_finish(partials, deg_all.T, h_self, W_self, W_neigh, bm)


# speedup vs baseline: 1.1327x; 1.1327x over previous
import functools

import jax
import jax.numpy as jnp
from jax import lax
from jax.experimental import pallas as pl
from jax.experimental.pallas import tpu as pltpu
from jax.experimental.pallas import tpu_sc as plsc

NC = 2
NS = 16
NW = NC * NS
L = 16
C = 64

NBUF = 4
DG = 2
DS = NBUF - DG
SLAB = 8
SUPER = 2


def _sc_aggregate(h_pad, src3d, dst3d, n_pad, d):
    rows_per_tile = n_pad // NS
    zchunks = rows_per_tile // C
    cpw = src3d.shape[1]
    nslab = cpw // SLAB
    nmain = nslab * SLAB
    assert nmain == cpw and nslab % SUPER == 0
    mesh = plsc.VectorSubcoreMesh(core_axis_name="c", subcore_axis_name="s")

    @functools.partial(
        pl.kernel,
        out_type=(
            jax.ShapeDtypeStruct((NC, n_pad, d), jnp.float32),
            jax.ShapeDtypeStruct((NW, n_pad), jnp.float32),
        ),
        mesh=mesh,
        compiler_params=pltpu.CompilerParams(needs_layout_passes=False),
        scratch_types=[
            [pltpu.VMEM((SLAB, C), jnp.int32) for _ in range(2)],
            [pltpu.VMEM((SLAB, C), jnp.int32) for _ in range(2)],
            [pltpu.VMEM((C, d), jnp.float32) for _ in range(NBUF)],
            pltpu.VMEM((n_pad,), jnp.float32),
            pltpu.VMEM_SHARED((n_pad, d), jnp.float32),
            [pltpu.SemaphoreType.DMA for _ in range(NBUF)],
            [pltpu.SemaphoreType.DMA for _ in range(NBUF)],
            [pltpu.SemaphoreType.DMA for _ in range(2)],
        ],
    )
    def agg(h_ref, src_ref, dst_ref, feat_ref, deg_ref, sslab, dslab, rows,
            degs, acc, gsem, ssem, lsem):
        cid = lax.axis_index("c")
        sid = lax.axis_index("s")
        wid = cid * NS + sid
        ones16 = jnp.full((L,), 1.0, jnp.float32)

        zvec = jnp.zeros((L,), jnp.float32)

        def zrow(i, carry):
            for j in range(d // L):
                rows[0][i, pl.ds(j * L, L)] = zvec
            return carry

        lax.fori_loop(0, C, zrow, 0)

        def zdeg(i, carry):
            for j in range(8):
                degs[pl.ds(i * 8 * L + j * L, L)] = zvec
            return carry

        lax.fori_loop(0, n_pad // (8 * L), zdeg, 0)

        r0 = sid * rows_per_tile

        def zacc(k, carry):
            pltpu.sync_copy(rows[0], acc.at[pl.ds(r0 + k * C, C)])
            return carry

        lax.fori_loop(0, zchunks, zacc, 0)
        plsc.subcore_barrier()

        def slab_copy_start(s, ring):
            pltpu.async_copy(src_ref.at[wid, pl.ds(s * SLAB, SLAB)],
                             sslab[ring], lsem[ring])
            pltpu.async_copy(dst_ref.at[wid, pl.ds(s * SLAB, SLAB)],
                             dslab[ring], lsem[ring])

        def slab_copy_wait(s, ring):
            pltpu.make_async_copy(src_ref.at[wid, pl.ds(s * SLAB, SLAB)],
                                  sslab[ring], lsem[ring]).wait()
            pltpu.make_async_copy(dst_ref.at[wid, pl.ds(s * SLAB, SLAB)],
                                  dslab[ring], lsem[ring]).wait()

        def gather(ring, k, b):
            pltpu.async_copy(h_ref.at[sslab[ring].at[k]], rows[b], gsem[b])

        def deg_count(ring, k):
            for g in range(C // L):
                dvec = dslab[ring][k, pl.ds(g * L, L)]
                plsc.addupdate_scatter(degs, [dvec], ones16)

        if nmain > 0:
            slab_copy_start(0, 0)
            slab_copy_wait(0, 0)
            for jj in range(DG):
                gather(0, jj, jj % NBUF)

            def super_body(p, carry):
                for half in range(SUPER):
                    s = p * SUPER + half
                    for k in range(SLAB):
                        j = s * SLAB + k
                        b = (SLAB * half + k) % NBUF
                        bg = (SLAB * half + k + DG) % NBUF

                        pltpu.make_async_copy(
                            h_ref.at[sslab[half].at[k]], rows[b], gsem[b]
                        ).wait()
                        pltpu.async_copy(rows[b], acc.at[dslab[half].at[k]],
                                         ssem[b], add=True)
                        deg_count(half, k)

                        if k == 2:
                            @pl.when(s + 1 < nslab)
                            def _():
                                slab_copy_start(s + 1, 1 - half)

                        @pl.when(j >= DS)
                        def _():
                            pltpu.make_async_copy(
                                rows[bg], acc.at[dslab[half].at[k]], ssem[bg]
                            ).wait()

                        if k == SLAB - DG:
                            @pl.when(s + 1 < nslab)
                            def _():
                                slab_copy_wait(s + 1, 1 - half)
                        if k < SLAB - DG:
                            gather(half, k + DG, bg)
                        else:
                            @pl.when(s + 1 < nslab)
                            def _():
                                gather(1 - half, k + DG - SLAB, bg)
                return carry

            lax.fori_loop(0, nslab // SUPER, super_body, 0)

            for j in range(nmain - DS, nmain):
                b = j % NBUF
                pltpu.make_async_copy(
                    rows[b], acc.at[dslab[0].at[0]], ssem[b]
                ).wait()

        plsc.subcore_barrier()

        pltpu.sync_copy(acc.at[pl.ds(r0, rows_per_tile)],
                        feat_ref.at[cid, pl.ds(r0, rows_per_tile)])
        pltpu.sync_copy(degs, deg_ref.at[wid])

    return agg(h_pad, src3d, dst3d)


def _tc_finish(partials, deg_all, h_self, W_self, W_neigh, bm):
    n, d = h_self.shape
    out = W_self.shape[1]

    def body(p_ref, dg_ref, hs_ref, ws_ref, wn_ref, o_ref):
        deg = jnp.sum(dg_ref[...], axis=1, keepdims=True)
        neigh = (p_ref[0] + p_ref[1]) / jnp.maximum(deg, 1.0)
        z = (jnp.dot(hs_ref[...], ws_ref[...], preferred_element_type=jnp.float32)
             + jnp.dot(neigh, wn_ref[...], preferred_element_type=jnp.float32))
        z = jnp.maximum(z, 0.0)
        nrm = jnp.sqrt(jnp.sum(z * z, axis=1, keepdims=True))
        nrm = jnp.where(nrm == 0.0, 1.0, nrm)
        o_ref[...] = z / nrm

    return pl.pallas_call(
        body,
        grid=(n // bm,),
        in_specs=[
            pl.BlockSpec((NC, bm, d), lambda i: (0, i, 0)),
            pl.BlockSpec((bm, NW), lambda i: (i, 0)),
            pl.BlockSpec((bm, d), lambda i: (i, 0)),
            pl.BlockSpec((d, out), lambda i: (0, 0)),
            pl.BlockSpec((d, out), lambda i: (0, 0)),
        ],
        out_specs=pl.BlockSpec((bm, out), lambda i: (i, 0)),
        out_shape=jax.ShapeDtypeStruct((n, out), jnp.float32),
    )(partials, deg_all, h_self, W_self, W_neigh)


def kernel(h_neigh, h_self, edge_index, W_neigh, W_self):
    n, d = h_neigh.shape
    e = edge_index.shape[1]

    n_pad = ((n + NS * C - 1) // (NS * C)) * (NS * C)
    if n_pad == n:
        n_pad += NS * C

    step = NW * C * SLAB * SUPER
    e_pad = ((e + step - 1) // step) * step
    cpw = e_pad // (NW * C)

    src = edge_index[0].astype(jnp.int32)
    dst = edge_index[1].astype(jnp.int32)
    pad = e_pad - e
    if pad:
        ar = jnp.arange(pad, dtype=jnp.int32)
        src = jnp.concatenate([src, ar % n])
        dst = jnp.concatenate([dst, n + ar % (n_pad - n)])
    src3d = src.reshape(NW, cpw, C)
    dst3d = dst.reshape(NW, cpw, C)

    partials, deg_all = _sc_aggregate(h_neigh, src3d, dst3d, n_pad, d)

    bm = next(b for b in (1000, 400, 200, 100, 50, 25, 10, 5, 1)
              if n % b == 0 and (b % 8 == 0 or b == n))
    return _tc_finish(partials, deg_all.T, h_self, W_self, W_neigh, bm)


# speedup vs baseline: 1.1773x; 1.0394x over previous
import functools

import jax
import jax.numpy as jnp
from jax import lax
from jax.experimental import pallas as pl
from jax.experimental.pallas import tpu as pltpu
from jax.experimental.pallas import tpu_sc as plsc

NC = 2
NS = 16
NW = NC * NS
L = 16
C = 64

NBUF = 4
DG = 2
DS = NBUF - DG
SLAB = 8
SUPER = 2


def _sc_aggregate(h_pad, sd4, n_pad, d):
    rows_per_tile = n_pad // NS
    zchunks = rows_per_tile // C
    cpw = sd4.shape[2]
    nslab = cpw // SLAB
    nmain = nslab * SLAB
    assert nmain == cpw and nslab % SUPER == 0
    mesh = plsc.VectorSubcoreMesh(core_axis_name="c", subcore_axis_name="s")

    @functools.partial(
        pl.kernel,
        out_type=(
            jax.ShapeDtypeStruct((NC, n_pad, d), jnp.float32),
            jax.ShapeDtypeStruct((NW, n_pad), jnp.float32),
        ),
        mesh=mesh,
        compiler_params=pltpu.CompilerParams(needs_layout_passes=False),
        scratch_types=[
            [pltpu.VMEM((SLAB, C), jnp.int32) for _ in range(2)],
            [pltpu.VMEM((SLAB, C), jnp.int32) for _ in range(2)],
            [pltpu.VMEM((C, d), jnp.float32) for _ in range(NBUF)],
            pltpu.VMEM((n_pad,), jnp.float32),
            pltpu.VMEM_SHARED((n_pad, d), jnp.float32),
            [pltpu.SemaphoreType.DMA for _ in range(NBUF)],
            [pltpu.SemaphoreType.DMA for _ in range(NBUF)],
            [pltpu.SemaphoreType.DMA for _ in range(2)],
        ],
    )
    def agg(h_ref, sd_ref, feat_ref, deg_ref, sslab, dslab, rows,
            degs, acc, gsem, ssem, lsem):
        cid = lax.axis_index("c")
        sid = lax.axis_index("s")
        wid = cid * NS + sid
        ones16 = jnp.full((L,), 1.0, jnp.float32)

        zvec = jnp.zeros((L,), jnp.float32)

        def slab_copy_start(s, ring):
            pltpu.async_copy(sd_ref.at[0, wid, pl.ds(s * SLAB, SLAB)],
                             sslab[ring], lsem[ring])
            pltpu.async_copy(sd_ref.at[1, wid, pl.ds(s * SLAB, SLAB)],
                             dslab[ring], lsem[ring])

        def slab_copy_wait(s, ring):
            pltpu.make_async_copy(sd_ref.at[0, wid, pl.ds(s * SLAB, SLAB)],
                                  sslab[ring], lsem[ring]).wait()
            pltpu.make_async_copy(sd_ref.at[1, wid, pl.ds(s * SLAB, SLAB)],
                                  dslab[ring], lsem[ring]).wait()

        def zrow(i, carry):
            for j in range(d // L):
                rows[0][i, pl.ds(j * L, L)] = zvec
            return carry

        lax.fori_loop(0, C, zrow, 0)
        if nmain > 0:
            slab_copy_start(0, 0)

        r0 = sid * rows_per_tile
        for k in range(zchunks):
            pltpu.async_copy(rows[0], acc.at[pl.ds(r0 + k * C, C)], ssem[0])

        def zdeg(i, carry):
            for j in range(8):
                degs[pl.ds(i * 8 * L + j * L, L)] = zvec
            return carry

        lax.fori_loop(0, n_pad // (8 * L), zdeg, 0)

        for k in range(zchunks):
            pltpu.make_async_copy(rows[0], acc.at[pl.ds(r0 + k * C, C)],
                                  ssem[0]).wait()
        plsc.subcore_barrier()

        def gather(ring, k, b):
            pltpu.async_copy(h_ref.at[sslab[ring].at[k]], rows[b], gsem[b])

        def deg_count(ring, k):
            for g in range(C // L):
                dvec = dslab[ring][k, pl.ds(g * L, L)]
                plsc.addupdate_scatter(degs, [dvec], ones16)

        if nmain > 0:
            slab_copy_wait(0, 0)
            for jj in range(DG):
                gather(0, jj, jj % NBUF)

            def super_body(p, carry):
                for half in range(SUPER):
                    s = p * SUPER + half
                    for k in range(SLAB):
                        j = s * SLAB + k
                        b = (SLAB * half + k) % NBUF
                        bg = (SLAB * half + k + DG) % NBUF

                        pltpu.make_async_copy(
                            h_ref.at[sslab[half].at[k]], rows[b], gsem[b]
                        ).wait()
                        pltpu.async_copy(rows[b], acc.at[dslab[half].at[k]],
                                         ssem[b], add=True)
                        deg_count(half, k)

                        if k == 2:
                            @pl.when(s + 1 < nslab)
                            def _():
                                slab_copy_start(s + 1, 1 - half)

                        @pl.when(j >= DS)
                        def _():
                            pltpu.make_async_copy(
                                rows[bg], acc.at[dslab[half].at[k]], ssem[bg]
                            ).wait()

                        if k == SLAB - DG:
                            @pl.when(s + 1 < nslab)
                            def _():
                                slab_copy_wait(s + 1, 1 - half)
                        if k < SLAB - DG:
                            gather(half, k + DG, bg)
                        else:
                            @pl.when(s + 1 < nslab)
                            def _():
                                gather(1 - half, k + DG - SLAB, bg)
                return carry

            lax.fori_loop(0, nslab // SUPER, super_body, 0)

            for j in range(nmain - DS, nmain):
                b = j % NBUF
                pltpu.make_async_copy(
                    rows[b], acc.at[dslab[0].at[0]], ssem[b]
                ).wait()

        plsc.subcore_barrier()

        pltpu.sync_copy(acc.at[pl.ds(r0, rows_per_tile)],
                        feat_ref.at[cid, pl.ds(r0, rows_per_tile)])
        pltpu.sync_copy(degs, deg_ref.at[wid])

    return agg(h_pad, sd4)


def _tc_self(h_self, W_self, bm):
    n, d = h_self.shape
    out = W_self.shape[1]

    def body(hs_ref, ws_ref, o_ref):
        o_ref[...] = jnp.dot(hs_ref[...], ws_ref[...],
                             preferred_element_type=jnp.float32)

    return pl.pallas_call(
        body,
        grid=(n // bm,),
        in_specs=[
            pl.BlockSpec((bm, d), lambda i: (i, 0)),
            pl.BlockSpec((d, out), lambda i: (0, 0)),
        ],
        out_specs=pl.BlockSpec((bm, out), lambda i: (i, 0)),
        out_shape=jax.ShapeDtypeStruct((n, out), jnp.float32),
    )(h_self, W_self)


def _tc_finish(partials, deg_all, selfz, W_neigh, bm):
    n, out = selfz.shape
    d = W_neigh.shape[0]

    def body(p_ref, dg_ref, sz_ref, wn_ref, o_ref):
        deg = jnp.sum(dg_ref[...], axis=1, keepdims=True)
        neigh = (p_ref[0] + p_ref[1]) / jnp.maximum(deg, 1.0)
        z = sz_ref[...] + jnp.dot(neigh, wn_ref[...],
                                  preferred_element_type=jnp.float32)
        z = jnp.maximum(z, 0.0)
        nrm = jnp.sqrt(jnp.sum(z * z, axis=1, keepdims=True))
        nrm = jnp.where(nrm == 0.0, 1.0, nrm)
        o_ref[...] = z / nrm

    return pl.pallas_call(
        body,
        grid=(n // bm,),
        in_specs=[
            pl.BlockSpec((NC, bm, d), lambda i: (0, i, 0)),
            pl.BlockSpec((bm, NW), lambda i: (i, 0)),
            pl.BlockSpec((bm, out), lambda i: (i, 0)),
            pl.BlockSpec((d, out), lambda i: (0, 0)),
        ],
        out_specs=pl.BlockSpec((bm, out), lambda i: (i, 0)),
        out_shape=jax.ShapeDtypeStruct((n, out), jnp.float32),
    )(partials, deg_all, selfz, W_neigh)


def kernel(h_neigh, h_self, edge_index, W_neigh, W_self):
    n, d = h_neigh.shape
    e = edge_index.shape[1]

    n_pad = ((n + NS * C - 1) // (NS * C)) * (NS * C)
    if n_pad == n:
        n_pad += NS * C

    step = NW * C * SLAB * SUPER
    e_pad = ((e + step - 1) // step) * step
    cpw = e_pad // (NW * C)

    ei = edge_index.astype(jnp.int32)
    pad = e_pad - e
    if pad:
        ar = jnp.arange(pad, dtype=jnp.int32)
        fill = jnp.stack([ar % n, n + ar % (n_pad - n)])
        ei = jnp.concatenate([ei, fill], axis=1)
    sd4 = ei.reshape(2, NW, cpw, C)

    bm = next(b for b in (1000, 400, 200, 100, 50, 25, 10, 5, 1)
              if n % b == 0 and (b % 8 == 0 or b == n))
    selfz = _tc_self(h_self, W_self, bm)
    partials, deg_all = _sc_aggregate(h_neigh, sd4, n_pad, d)
    return _tc_finish(partials, deg_all.T, selfz, W_neigh, bm)


# speedup vs baseline: 1.3252x; 1.1256x over previous
import functools

import jax
import jax.numpy as jnp
from jax import lax
from jax.experimental import pallas as pl
from jax.experimental.pallas import tpu as pltpu
from jax.experimental.pallas import tpu_sc as plsc

NC = 2
NS = 16
NW = NC * NS
L = 16
C = 64

NBUF = 4
DG = 3
DS = NBUF - DG
SLAB = 8
SUPER = 2


def _sc_aggregate(h_pad, sd4, n_pad, d):
    rows_per_tile = n_pad // NS
    zchunks = rows_per_tile // C
    cpw = sd4.shape[2]
    nslab = cpw // SLAB
    nmain = nslab * SLAB
    assert nmain == cpw and nslab % SUPER == 0
    mesh = plsc.VectorSubcoreMesh(core_axis_name="c", subcore_axis_name="s")

    @functools.partial(
        pl.kernel,
        out_type=(
            jax.ShapeDtypeStruct((NC, n_pad, d), jnp.float32),
            jax.ShapeDtypeStruct((NW, n_pad), jnp.float32),
        ),
        mesh=mesh,
        compiler_params=pltpu.CompilerParams(needs_layout_passes=False),
        scratch_types=[
            [pltpu.VMEM((SLAB, C), jnp.int32) for _ in range(2)],
            [pltpu.VMEM((SLAB, C), jnp.int32) for _ in range(2)],
            [pltpu.VMEM((C, d), jnp.float32) for _ in range(NBUF)],
            pltpu.VMEM((n_pad,), jnp.float32),
            pltpu.VMEM_SHARED((n_pad, d), jnp.float32),
            [pltpu.SemaphoreType.DMA for _ in range(NBUF)],
            [pltpu.SemaphoreType.DMA for _ in range(NBUF)],
            [pltpu.SemaphoreType.DMA for _ in range(2)],
        ],
    )
    def agg(h_ref, sd_ref, feat_ref, deg_ref, sslab, dslab, rows,
            degs, acc, gsem, ssem, lsem):
        cid = lax.axis_index("c")
        sid = lax.axis_index("s")
        wid = cid * NS + sid
        ones16 = jnp.full((L,), 1.0, jnp.float32)

        zvec = jnp.zeros((L,), jnp.float32)

        def slab_copy_start(s, ring):
            pltpu.async_copy(sd_ref.at[0, wid, pl.ds(s * SLAB, SLAB)],
                             sslab[ring], lsem[ring])
            pltpu.async_copy(sd_ref.at[1, wid, pl.ds(s * SLAB, SLAB)],
                             dslab[ring], lsem[ring])

        def slab_copy_wait(s, ring):
            pltpu.make_async_copy(sd_ref.at[0, wid, pl.ds(s * SLAB, SLAB)],
                                  sslab[ring], lsem[ring]).wait()
            pltpu.make_async_copy(sd_ref.at[1, wid, pl.ds(s * SLAB, SLAB)],
                                  dslab[ring], lsem[ring]).wait()

        def zrow(i, carry):
            for j in range(d // L):
                rows[0][i, pl.ds(j * L, L)] = zvec
            return carry

        lax.fori_loop(0, C, zrow, 0)
        if nmain > 0:
            slab_copy_start(0, 0)

        r0 = sid * rows_per_tile
        for k in range(zchunks):
            pltpu.async_copy(rows[0], acc.at[pl.ds(r0 + k * C, C)], ssem[0])

        def zdeg(i, carry):
            for j in range(8):
                degs[pl.ds(i * 8 * L + j * L, L)] = zvec
            return carry

        lax.fori_loop(0, n_pad // (8 * L), zdeg, 0)

        for k in range(zchunks):
            pltpu.make_async_copy(rows[0], acc.at[pl.ds(r0 + k * C, C)],
                                  ssem[0]).wait()
        plsc.subcore_barrier()

        def gather(ring, k, b):
            pltpu.async_copy(h_ref.at[sslab[ring].at[k]], rows[b], gsem[b])

        def deg_count(ring, k):
            for g in range(C // L):
                dvec = dslab[ring][k, pl.ds(g * L, L)]
                plsc.addupdate_scatter(degs, [dvec], ones16)

        if nmain > 0:
            slab_copy_wait(0, 0)
            for jj in range(DG):
                gather(0, jj, jj % NBUF)

            def super_body(p, carry):
                for half in range(SUPER):
                    s = p * SUPER + half
                    for k in range(SLAB):
                        j = s * SLAB + k
                        b = (SLAB * half + k) % NBUF
                        bg = (SLAB * half + k + DG) % NBUF

                        pltpu.make_async_copy(
                            h_ref.at[sslab[half].at[k]], rows[b], gsem[b]
                        ).wait()
                        pltpu.async_copy(rows[b], acc.at[dslab[half].at[k]],
                                         ssem[b], add=True)
                        deg_count(half, k)

                        if k == 2:
                            @pl.when(s + 1 < nslab)
                            def _():
                                slab_copy_start(s + 1, 1 - half)

                        @pl.when(j >= DS)
                        def _():
                            pltpu.make_async_copy(
                                rows[bg], acc.at[dslab[half].at[k]], ssem[bg]
                            ).wait()

                        if k == SLAB - DG:
                            @pl.when(s + 1 < nslab)
                            def _():
                                slab_copy_wait(s + 1, 1 - half)
                        if k < SLAB - DG:
                            gather(half, k + DG, bg)
                        else:
                            @pl.when(s + 1 < nslab)
                            def _():
                                gather(1 - half, k + DG - SLAB, bg)
                return carry

            lax.fori_loop(0, nslab // SUPER, super_body, 0)

            for j in range(nmain - DS, nmain):
                b = j % NBUF
                pltpu.make_async_copy(
                    rows[b], acc.at[dslab[0].at[0]], ssem[b]
                ).wait()

        plsc.subcore_barrier()

        pltpu.sync_copy(acc.at[pl.ds(r0, rows_per_tile)],
                        feat_ref.at[cid, pl.ds(r0, rows_per_tile)])
        pltpu.sync_copy(degs, deg_ref.at[wid])

    return agg(h_pad, sd4)


def _tc_self(h_self, W_self, bm):
    n, d = h_self.shape
    out = W_self.shape[1]

    def body(hs_ref, ws_ref, o_ref):
        o_ref[...] = jnp.dot(hs_ref[...], ws_ref[...],
                             preferred_element_type=jnp.float32)

    return pl.pallas_call(
        body,
        grid=(n // bm,),
        in_specs=[
            pl.BlockSpec((bm, d), lambda i: (i, 0)),
            pl.BlockSpec((d, out), lambda i: (0, 0)),
        ],
        out_specs=pl.BlockSpec((bm, out), lambda i: (i, 0)),
        out_shape=jax.ShapeDtypeStruct((n, out), jnp.float32),
    )(h_self, W_self)


def _tc_finish(partials, deg_all, selfz, W_neigh, bm):
    n, out = selfz.shape
    d = W_neigh.shape[0]

    def body(p_ref, dg_ref, sz_ref, wn_ref, o_ref):
        deg = jnp.sum(dg_ref[...], axis=1, keepdims=True)
        neigh = (p_ref[0] + p_ref[1]) / jnp.maximum(deg, 1.0)
        z = sz_ref[...] + jnp.dot(neigh, wn_ref[...],
                                  preferred_element_type=jnp.float32)
        z = jnp.maximum(z, 0.0)
        nrm = jnp.sqrt(jnp.sum(z * z, axis=1, keepdims=True))
        nrm = jnp.where(nrm == 0.0, 1.0, nrm)
        o_ref[...] = z / nrm

    return pl.pallas_call(
        body,
        grid=(n // bm,),
        in_specs=[
            pl.BlockSpec((NC, bm, d), lambda i: (0, i, 0)),
            pl.BlockSpec((bm, NW), lambda i: (i, 0)),
            pl.BlockSpec((bm, out), lambda i: (i, 0)),
            pl.BlockSpec((d, out), lambda i: (0, 0)),
        ],
        out_specs=pl.BlockSpec((bm, out), lambda i: (i, 0)),
        out_shape=jax.ShapeDtypeStruct((n, out), jnp.float32),
    )(partials, deg_all, selfz, W_neigh)


def kernel(h_neigh, h_self, edge_index, W_neigh, W_self):
    n, d = h_neigh.shape
    e = edge_index.shape[1]

    n_pad = ((n + NS * C - 1) // (NS * C)) * (NS * C)
    if n_pad == n:
        n_pad += NS * C

    step = NW * C * SLAB * SUPER
    e_pad = ((e + step - 1) // step) * step
    cpw = e_pad // (NW * C)

    ei = edge_index.astype(jnp.int32)
    pad = e_pad - e
    if pad:
        ar = jnp.arange(pad, dtype=jnp.int32)
        fill = jnp.stack([ar % n, n + ar % (n_pad - n)])
        ei = jnp.concatenate([ei, fill], axis=1)
    sd4 = ei.reshape(2, NW, cpw, C)

    bm = next(b for b in (1000, 400, 200, 100, 50, 25, 10, 5, 1)
              if n % b == 0 and (b % 8 == 0 or b == n))
    selfz = _tc_self(h_self, W_self, bm)
    partials, deg_all = _sc_aggregate(h_neigh, sd4, n_pad, d)
    return _tc_finish(partials, deg_all.T, selfz, W_neigh, bm)
